# baseline (speedup 1.0000x reference)
import functools

import jax
import jax.numpy as jnp
from jax import lax
from jax.experimental import pallas as pl
from jax.experimental.pallas import tpu as pltpu
from jax.experimental.pallas import tpu_sc as plsc

N = 4096
NN = N * N
NC, NS = 2, 16
NW = NC * NS
CHUNK = NN // NW
NLANE = 16
BLK = CHUNK // NLANE
WIN = 512
NWIN = BLK // WIN
WELE = NLANE * WIN
NVREG = WELE // 16
NCHB = WELE // 128
RADIX = 1024
CT_SZ = RADIX * NW * NLANE


def _rowsq(v):
    v = v * v
    while v.shape[-1] > 1:
        h = v.shape[-1] // 2
        v = v[:, :h] + v[:, h:]
    return v[:, 0]


def _keys_body(x_blk_ref, x_ref, out_ref):
    xi = x_blk_ref[...]
    xa = x_ref[...]
    aa = _rowsq(xi)
    bb = _rowsq(xa)
    mm = jax.lax.dot_general(xi, xa, (((1,), (1,)), ((), ())))
    sq = aa[:, None] + bb[None, :] - 2.0 * mm
    sq = jnp.maximum(sq, 1e-12)
    sims = jnp.exp(-jnp.sqrt(sq) / x_ref.shape[-1])
    out_ref[...] = ~lax.bitcast_convert_type(sims, jnp.int32)


def _keys_matrix(x):
    n, d = x.shape
    br = 256
    return pl.pallas_call(
        _keys_body,
        grid=(n // br,),
        in_specs=[
            pl.BlockSpec((br, d), lambda i: (i, 0)),
            pl.BlockSpec((n, d), lambda i: (0, 0)),
        ],
        out_specs=pl.BlockSpec((br, n), lambda i: (i, 0)),
        out_shape=jax.ShapeDtypeStruct((n, n), jnp.int32),
    )(x, x)


def _wid():
    return lax.axis_index("s") * NC + lax.axis_index("c")


def _build_cidx(cidx, wid, iota):
    def bcidx(c, _):
        jj = c * 16 + iota
        dest = (jj >> 4) * (NW * NLANE) + wid * NLANE + (jj & 15)
        cidx[pl.ds(c * 16, 16)] = dest
        return 0
    lax.fori_loop(0, RADIX, bcidx, 0)


def _stage(keys_hbm, kwin, base, sem):
    hs = [
        pltpu.async_copy(keys_hbm.at[pl.ds(base + l * BLK, WIN)],
                         kwin.at[l], sem)
        for l in range(NLANE)
    ]
    for h in hs:
        h.wait()


def _transpose(kwin, kt, iota):
    for l in range(NLANE):
        def btr(tv, _, l=l):
            v = kwin[l, pl.ds(tv * 16, 16)]
            plsc.store_scatter(kt, [(tv * 16 + iota) * 16 + l], v)
            return 0
        lax.fori_loop(0, WIN // 16, btr, 0)


def _phase_a(keys, shift):
    mesh = plsc.VectorSubcoreMesh(core_axis_name="c", subcore_axis_name="s")

    @functools.partial(
        pl.kernel,
        out_type=jax.ShapeDtypeStruct((CT_SZ,), jnp.int32),
        mesh=mesh,
        compiler_params=pltpu.CompilerParams(needs_layout_passes=False),
        scratch_types=[
            pltpu.VMEM((NLANE, WIN), jnp.int32),
            pltpu.VMEM((WELE,), jnp.int32),
            pltpu.VMEM((RADIX * NLANE,), jnp.int32),
            pltpu.VMEM((RADIX * NLANE,), jnp.int32),
            pltpu.SemaphoreType.DMA,
            pltpu.SemaphoreType.DMA,
        ],
    )
    def k(keys_hbm, ct_hbm, kwin, kt, hist, cidx, sems, semo):
        wid = _wid()
        iota = lax.iota(jnp.int32, 16)
        ones = jnp.ones((16,), jnp.int32)
        _build_cidx(cidx, wid, iota)

        def bz(h, _):
            hist[pl.ds(h * 16, 16)] = jnp.zeros((16,), jnp.int32)
            return 0
        lax.fori_loop(0, RADIX, bz, 0)

        def bwin(tw, _):
            base = wid * CHUNK + tw * WIN
            _stage(keys_hbm, kwin, base, sems)
            _transpose(kwin, kt, iota)

            def bh(v, _):
                kv = kt[pl.ds(v * 16, 16)]
                digit = (kv >> shift) & (RADIX - 1)
                plsc.addupdate_scatter(hist, [digit * 16 + iota], ones)
                return 0
            lax.fori_loop(0, NVREG, bh, 0)
            return 0
        lax.fori_loop(0, NWIN, bwin, 0)

        pltpu.async_copy(hist, ct_hbm.at[cidx], semo).wait()

    return k(keys)


def _phase_b(ct):
    mesh = plsc.VectorSubcoreMesh(core_axis_name="c", subcore_axis_name="s")

    @functools.partial(
        pl.kernel,
        out_type=jax.ShapeDtypeStruct((CT_SZ,), jnp.int32),
        mesh=mesh,
        compiler_params=pltpu.CompilerParams(needs_layout_passes=False),
        scratch_types=[
            pltpu.VMEM((WELE,), jnp.int32),
            pltpu.VMEM((WELE,), jnp.int32),
        ],
    )
    def k(ct_hbm, ot_hbm, buf, obuf):
        wid = _wid()

        @pl.when(wid == 0)
        def _():
            def bwin(w, carry):
                pltpu.sync_copy(ct_hbm.at[pl.ds(w * WELE, WELE)], buf)

                def bv(v, c2):
                    kv = buf[pl.ds(v * 16, 16)]
                    inc = plsc.cumsum(kv)
                    obuf[pl.ds(v * 16, 16)] = (inc - kv) + c2
                    return c2 + jnp.sum(kv)
                c2 = lax.fori_loop(0, NVREG, bv, carry)
                pltpu.sync_copy(obuf, ot_hbm.at[pl.ds(w * WELE, WELE)])
                return c2
            lax.fori_loop(0, CT_SZ // WELE, bwin, jnp.int32(0))

    return k(ct)


def _phase_c(keys, pay, ot, shift, last):
    mesh = plsc.VectorSubcoreMesh(core_axis_name="c", subcore_axis_name="s")
    first = pay is None

    if last:
        out_type = (jax.ShapeDtypeStruct((NN,), jnp.float32),
                    jax.ShapeDtypeStruct((2 * NN,), jnp.int32))
    else:
        out_type = (jax.ShapeDtypeStruct((NN,), jnp.int32),
                    jax.ShapeDtypeStruct((NN,), jnp.int32))

    scratch = [
        pltpu.VMEM((NLANE, WIN), jnp.int32),
        pltpu.VMEM((WELE,), jnp.int32),
        pltpu.VMEM((NLANE, WIN), jnp.int32),
        pltpu.VMEM((WELE,), jnp.int32),
        pltpu.VMEM((RADIX * NLANE,), jnp.int32),
        pltpu.VMEM((RADIX * NLANE,), jnp.int32),
        pltpu.VMEM((WELE,), jnp.int32),
        pltpu.VMEM((WELE,), jnp.int32),
        pltpu.VMEM((WELE,), jnp.float32),
        pltpu.VMEM((WELE,), jnp.int32),
        pltpu.VMEM((WELE,), jnp.int32),
        pltpu.SemaphoreType.DMA,
        pltpu.SemaphoreType.DMA,
    ]

    def body(*refs):
        if first:
            keys_hbm, ot_hbm = refs[0], refs[1]
            pay_hbm = None
            o1, o2 = refs[2], refs[3]
            rest = refs[4:]
        else:
            keys_hbm, pay_hbm, ot_hbm = refs[0], refs[1], refs[2]
            o1, o2 = refs[3], refs[4]
            rest = refs[5:]
        (kwin, kt, pwin, pt, cursor, cidx, dbuf, cdbuf, vbuf, rbuf, cbuf,
         sems, semo) = rest

        wid = _wid()
        iota = lax.iota(jnp.int32, 16)
        ones = jnp.ones((16,), jnp.int32)
        _build_cidx(cidx, wid, iota)
        pltpu.async_copy(ot_hbm.at[cidx], cursor, semo).wait()

        def bwin(tw, _):
            base = wid * CHUNK + tw * WIN
            _stage(keys_hbm, kwin, base, sems)
            _transpose(kwin, kt, iota)
            if not first:
                _stage(pay_hbm, pwin, base, sems)
                _transpose(pwin, pt, iota)

            def bv(v, _):
                sl = pl.ds(v * 16, 16)
                kv = kt[sl]
                digit = (kv >> shift) & (RADIX - 1)
                bidx = digit * 16 + iota
                old = plsc.load_gather(cursor, [bidx])
                plsc.addupdate_scatter(cursor, [bidx], ones)
                dbuf[sl] = old
                if first:
                    pt[sl] = wid * CHUNK + iota * BLK + tw * WIN + v
                if last:
                    cdbuf[sl] = old + NN
                    vbuf[sl] = plsc.bitcast(~kv, jnp.float32)
                    pv = pt[sl]
                    rbuf[sl] = pv >> 12
                    cbuf[sl] = pv & (N - 1)
                return 0
            lax.fori_loop(0, NVREG, bv, 0)

            if not last:
                hs = [pltpu.async_copy(kt, o1.at[dbuf], semo),
                      pltpu.async_copy(pt, o2.at[dbuf], semo)]
            else:
                hs = [pltpu.async_copy(vbuf, o1.at[dbuf], semo),
                      pltpu.async_copy(rbuf, o2.at[dbuf], semo),
                      pltpu.async_copy(cbuf, o2.at[cdbuf], semo)]
            for h in hs:
                h.wait()
            return 0
        lax.fori_loop(0, NWIN, bwin, 0)

    k = functools.partial(
        pl.kernel, out_type=out_type, mesh=mesh,
        compiler_params=pltpu.CompilerParams(needs_layout_passes=False),
        scratch_types=scratch)(body)
    if first:
        return k(keys, ot)
    return k(keys, pay, ot)


def kernel(x):
    keys0 = _keys_matrix(x).reshape(NN)

    ct1 = _phase_a(keys0, 0)
    ot1 = _phase_b(ct1)
    k1, p1 = _phase_c(keys0, None, ot1, 0, last=False)

    ct2 = _phase_a(k1, 10)
    ot2 = _phase_b(ct2)
    k2, p2 = _phase_c(k1, p1, ot2, 10, last=False)

    ct3 = _phase_a(k2, 20)
    ot3 = _phase_b(ct3)
    w, ei = _phase_c(k2, p2, ot3, 20, last=True)

    return ei.reshape(2, NN), w


# speedup vs baseline: 1.2007x; 1.2007x over previous
import functools

import jax
import jax.numpy as jnp
from jax import lax
from jax.experimental import pallas as pl
from jax.experimental.pallas import tpu as pltpu
from jax.experimental.pallas import tpu_sc as plsc

N = 4096
NN = N * N
NC, NS = 2, 16
NW = NC * NS
CHUNK = NN // NW
NLANE = 16
BLK = CHUNK // NLANE
WIN = 512
NWIN = BLK // WIN
WELE = NLANE * WIN
NVREG = WELE // 16
RADIX = 1024
CT_SZ = RADIX * NW * NLANE


def _rowsq(v):
    v = v * v
    while v.shape[-1] > 1:
        h = v.shape[-1] // 2
        v = v[:, :h] + v[:, h:]
    return v[:, 0]


def _keys_body(x_blk_ref, x_ref, out_ref):
    xi = x_blk_ref[...]
    xa = x_ref[...]
    aa = _rowsq(xi)
    bb = _rowsq(xa)
    mm = jax.lax.dot_general(xi, xa, (((1,), (1,)), ((), ())))
    sq = aa[:, None] + bb[None, :] - 2.0 * mm
    sq = jnp.maximum(sq, 1e-12)
    sims = jnp.exp(-jnp.sqrt(sq) / x_ref.shape[-1])
    out_ref[...] = ~lax.bitcast_convert_type(sims, jnp.int32)


def _keys_matrix(x):
    n, d = x.shape
    br = 256
    return pl.pallas_call(
        _keys_body,
        grid=(n // br,),
        in_specs=[
            pl.BlockSpec((br, d), lambda i: (i, 0)),
            pl.BlockSpec((n, d), lambda i: (0, 0)),
        ],
        out_specs=pl.BlockSpec((br, n), lambda i: (i, 0)),
        out_shape=jax.ShapeDtypeStruct((n, n), jnp.int32),
    )(x, x)


def _unpack_body(ps_ref, ei_ref):
    p = ps_ref[...]
    ei_ref[0] = p >> 12
    ei_ref[1] = p & (N - 1)


def _unpack(ps):
    br = 256
    ei = pl.pallas_call(
        _unpack_body,
        grid=(N // br,),
        in_specs=[pl.BlockSpec((br, N), lambda i: (i, 0))],
        out_specs=pl.BlockSpec((2, br, N), lambda i: (0, i, 0)),
        out_shape=jax.ShapeDtypeStruct((2, N, N), jnp.int32),
    )(ps.reshape(N, N))
    return ei.reshape(2, NN)


def _wid():
    return lax.axis_index("s") * NC + lax.axis_index("c")


def _build_cidx(cidx, wid, iota):
    def bcidx(c, _):
        jj = c * 16 + iota
        dest = (jj >> 4) * (NW * NLANE) + wid * NLANE + (jj & 15)
        cidx[pl.ds(c * 16, 16)] = dest
        return 0
    lax.fori_loop(0, RADIX, bcidx, 0)


def _stage(keys_hbm, kwin, base, sem):
    hs = [
        pltpu.async_copy(keys_hbm.at[pl.ds(base + l * BLK, WIN)],
                         kwin.at[l], sem)
        for l in range(NLANE)
    ]
    for h in hs:
        h.wait()


def _transpose(kwin, kt, iota):
    for l in range(NLANE):
        def btr(tv, _, l=l):
            v = kwin[l, pl.ds(tv * 16, 16)]
            plsc.store_scatter(kt, [(tv * 16 + iota) * 16 + l], v)
            return 0
        lax.fori_loop(0, WIN // 16, btr, 0)


def _phase_a(keys, shift):
    mesh = plsc.VectorSubcoreMesh(core_axis_name="c", subcore_axis_name="s")

    @functools.partial(
        pl.kernel,
        out_type=jax.ShapeDtypeStruct((CT_SZ,), jnp.int32),
        mesh=mesh,
        compiler_params=pltpu.CompilerParams(needs_layout_passes=False),
        scratch_types=[
            pltpu.VMEM((NLANE, WIN), jnp.int32),
            pltpu.VMEM((WELE,), jnp.int32),
            pltpu.VMEM((RADIX * NLANE,), jnp.int32),
            pltpu.VMEM((RADIX * NLANE,), jnp.int32),
            pltpu.SemaphoreType.DMA,
            pltpu.SemaphoreType.DMA,
        ],
    )
    def k(keys_hbm, ct_hbm, kwin, kt, hist, cidx, sems, semo):
        wid = _wid()
        iota = lax.iota(jnp.int32, 16)
        ones = jnp.ones((16,), jnp.int32)
        _build_cidx(cidx, wid, iota)

        def bz(h, _):
            hist[pl.ds(h * 16, 16)] = jnp.zeros((16,), jnp.int32)
            return 0
        lax.fori_loop(0, RADIX, bz, 0)

        def bwin(tw, _):
            base = wid * CHUNK + tw * WIN
            _stage(keys_hbm, kwin, base, sems)
            _transpose(kwin, kt, iota)

            def bh(v, _):
                kv = kt[pl.ds(v * 16, 16)]
                digit = (kv >> shift) & (RADIX - 1)
                plsc.addupdate_scatter(hist, [digit * 16 + iota], ones)
                return 0
            lax.fori_loop(0, NVREG, bh, 0)
            return 0
        lax.fori_loop(0, NWIN, bwin, 0)

        pltpu.async_copy(hist, ct_hbm.at[cidx], semo).wait()

    return k(keys)


def _phase_b(ct):
    mesh = plsc.VectorSubcoreMesh(core_axis_name="c", subcore_axis_name="s")

    @functools.partial(
        pl.kernel,
        out_type=jax.ShapeDtypeStruct((CT_SZ,), jnp.int32),
        mesh=mesh,
        compiler_params=pltpu.CompilerParams(needs_layout_passes=False),
        scratch_types=[
            pltpu.VMEM((WELE,), jnp.int32),
            pltpu.VMEM((WELE,), jnp.int32),
        ],
    )
    def k(ct_hbm, ot_hbm, buf, obuf):
        wid = _wid()

        @pl.when(wid == 0)
        def _():
            def bwin(w, carry):
                pltpu.sync_copy(ct_hbm.at[pl.ds(w * WELE, WELE)], buf)

                def bv(v, c2):
                    kv = buf[pl.ds(v * 16, 16)]
                    inc = plsc.cumsum(kv)
                    obuf[pl.ds(v * 16, 16)] = (inc - kv) + c2
                    return c2 + jnp.sum(kv)
                c2 = lax.fori_loop(0, NVREG, bv, carry)
                pltpu.sync_copy(obuf, ot_hbm.at[pl.ds(w * WELE, WELE)])
                return c2
            lax.fori_loop(0, CT_SZ // WELE, bwin, jnp.int32(0))

    return k(ct)


def _phase_c(keys, pay, ot, shift, last):
    mesh = plsc.VectorSubcoreMesh(core_axis_name="c", subcore_axis_name="s")
    first = pay is None

    if last:
        out_type = (jax.ShapeDtypeStruct((NN,), jnp.float32),
                    jax.ShapeDtypeStruct((NN,), jnp.int32))
    else:
        out_type = (jax.ShapeDtypeStruct((NN,), jnp.int32),
                    jax.ShapeDtypeStruct((NN,), jnp.int32))

    scratch = [
        pltpu.VMEM((NLANE, WIN), jnp.int32),
        pltpu.VMEM((NLANE, WIN), jnp.int32),
        pltpu.VMEM((RADIX * NLANE,), jnp.int32),
        pltpu.VMEM((RADIX * NLANE,), jnp.int32),
        pltpu.VMEM((WELE,), jnp.int32),
        pltpu.VMEM((WELE,), jnp.int32),
        pltpu.VMEM((WELE,), jnp.int32),
        pltpu.VMEM((WELE,), jnp.int32),
        pltpu.VMEM((WELE,), jnp.int32),
        pltpu.VMEM((WELE,), jnp.int32),
        pltpu.VMEM((WELE,), jnp.float32),
        pltpu.VMEM((WELE,), jnp.float32),
        pltpu.SemaphoreType.DMA,
        pltpu.SemaphoreType.DMA,
    ]

    def body(*refs):
        if first:
            keys_hbm, ot_hbm = refs[0], refs[1]
            pay_hbm = None
            o1, o2 = refs[2], refs[3]
            rest = refs[4:]
        else:
            keys_hbm, pay_hbm, ot_hbm = refs[0], refs[1], refs[2]
            o1, o2 = refs[3], refs[4]
            rest = refs[5:]
        (kwin, pwin, cursor, cidx, kt0, kt1, pt0, pt1, d0, d1, v0, v1,
         sems, semo) = rest
        kts, pts, dbufs, vbufs = (kt0, kt1), (pt0, pt1), (d0, d1), (v0, v1)

        wid = _wid()
        iota = lax.iota(jnp.int32, 16)
        ones = jnp.ones((16,), jnp.int32)
        _build_cidx(cidx, wid, iota)
        pltpu.async_copy(ot_hbm.at[cidx], cursor, semo).wait()

        def _drain():
            pltpu.make_async_copy(keys_hbm.at[pl.ds(0, WELE)],
                                  kt0, semo).wait()
            pltpu.make_async_copy(keys_hbm.at[pl.ds(0, WELE)],
                                  pt0, semo).wait()

        def bwin(half, _):
            for par in (0, 1):
                kt, pt, dbuf, vbuf = kts[par], pts[par], dbufs[par], vbufs[par]
                tw = half * 2 + par

                @pl.when(half > 0)
                def _():
                    _drain()

                base = wid * CHUNK + tw * WIN
                _stage(keys_hbm, kwin, base, sems)
                _transpose(kwin, kt, iota)
                if not first:
                    _stage(pay_hbm, pwin, base, sems)
                    _transpose(pwin, pt, iota)

                def bv(v, _, kt=kt, pt=pt, dbuf=dbuf, vbuf=vbuf, tw=tw):
                    sl = pl.ds(v * 16, 16)
                    kv = kt[sl]
                    digit = (kv >> shift) & (RADIX - 1)
                    bidx = digit * 16 + iota
                    old = plsc.load_gather(cursor, [bidx])
                    plsc.addupdate_scatter(cursor, [bidx], ones)
                    dbuf[sl] = old
                    if first:
                        pt[sl] = wid * CHUNK + iota * BLK + tw * WIN + v
                    if last:
                        vbuf[sl] = plsc.bitcast(~kv, jnp.float32)
                    return 0
                lax.fori_loop(0, NVREG, bv, 0)

                if not last:
                    pltpu.async_copy(kt, o1.at[dbuf], semo)
                else:
                    pltpu.async_copy(vbuf, o1.at[dbuf], semo)
                pltpu.async_copy(pt, o2.at[dbuf], semo)
            return 0
        lax.fori_loop(0, NWIN // 2, bwin, 0)
        _drain()
        _drain()

    k = functools.partial(
        pl.kernel, out_type=out_type, mesh=mesh,
        compiler_params=pltpu.CompilerParams(needs_layout_passes=False),
        scratch_types=scratch)(body)
    if first:
        return k(keys, ot)
    return k(keys, pay, ot)


def kernel(x):
    keys0 = _keys_matrix(x).reshape(NN)

    ct1 = _phase_a(keys0, 0)
    ot1 = _phase_b(ct1)
    k1, p1 = _phase_c(keys0, None, ot1, 0, last=False)

    ct2 = _phase_a(k1, 10)
    ot2 = _phase_b(ct2)
    k2, p2 = _phase_c(k1, p1, ot2, 10, last=False)

    ct3 = _phase_a(k2, 20)
    ot3 = _phase_b(ct3)
    w, ps = _phase_c(k2, p2, ot3, 20, last=True)

    return _unpack(ps), w
